# Initial kernel scaffold; baseline (speedup 1.0000x reference)
import jax
import jax.numpy as jnp
from jax.experimental import pallas as pl


def kernel(pl_x, pl_edge_index, pl_edge_attr, p_x, p_edge_index, p_edge_attr, l_x, l_edge_index, l_edge_attr, pl_params, p_params, l_params, mlp_params):
    raise NotImplementedError("write your pallas kernel here")


# speedup vs baseline: 36.7389x; 36.7389x over previous
import functools

import jax
import jax.numpy as jnp
from jax import lax
from jax.experimental import pallas as pl
from jax.experimental.pallas import tpu as pltpu
from jax.experimental.pallas import tpu_sc as plsc

F32 = jnp.float32
I32 = jnp.int32

NB_BIG = 10240
NB_L = 2048
CH_BIG = 79
CH_L = 16
EB = 16 * CH_BIG * 128
EL = 16 * CH_L * 128
R_BIG = NB_BIG // 16
R_L = NB_L // 16


def _proj_body(x_ref, wt_ref, c_ref, yz_ref):
    yz = lax.dot_general(wt_ref[0], x_ref[0], (((1,), (1,)), ((), ())),
                         preferred_element_type=F32,
                         precision=lax.Precision.HIGHEST)
    yz_ref[0] = yz + c_ref[0]


def _proj(x, wt, cc, wmap, bn):
    g_, nb, d_ = x.shape
    return pl.pallas_call(
        _proj_body,
        grid=(g_, nb // bn),
        in_specs=[
            pl.BlockSpec((1, bn, d_), lambda g, n: (g, n, 0)),
            pl.BlockSpec((1, 16, d_), lambda g, n: (wmap(g), 0, 0)),
            pl.BlockSpec((1, 16, 1), lambda g, n: (wmap(g), 0, 0)),
        ],
        out_specs=pl.BlockSpec((1, 16, bn), lambda g, n: (g, 0, n)),
        out_shape=jax.ShapeDtypeStruct((g_, 16, nb), F32),
    )(x, wt, cc)


def _sc_graphs(yz_b, ei_b, yz_l, ei_l, wpack, zcol):
    mesh = plsc.VectorSubcoreMesh(core_axis_name="c", subcore_axis_name="s")

    @functools.partial(
        pl.kernel,
        out_type=jax.ShapeDtypeStruct((24 * 16,), F32),
        mesh=mesh,
        scratch_types=[
            [pltpu.VMEM_SHARED((NB_BIG,), F32) for _ in range(5)],
            [pltpu.VMEM_SHARED((NB_BIG,), F32) for _ in range(6)],
            pltpu.VMEM_SHARED((256,), F32),
            pltpu.VMEM((128,), I32),
            pltpu.VMEM((128,), I32),
            pltpu.VMEM((128,), F32),
            [pltpu.VMEM((128,), F32) for _ in range(5)],
            [pltpu.VMEM((R_BIG,), F32) for _ in range(6)],
            [pltpu.VMEM((R_BIG,), F32) for _ in range(5)],
            [pltpu.VMEM((R_BIG,), F32) for _ in range(5)],
            [pltpu.VMEM((R_BIG,), F32) for _ in range(5)],
            pltpu.VMEM((R_BIG,), F32),
            pltpu.VMEM((16,), F32),
            pltpu.VMEM((16,), F32),
            pltpu.VMEM((256,), F32),
            pltpu.VMEM((352,), F32),
            pltpu.SemaphoreType.DMA,
        ],
    )
    def body(yz_b_ref, ei_b_ref, yz_l_ref, ei_l_ref, wpack_ref, z_ref,
             out_ref, tb, ac, part, sidx, didx, ones, gr,
             cb, zb, ub, vc, invb, eacc, erow, partv, wbuf, sem):
        cid = lax.axis_index("c")
        sid = lax.axis_index("s")
        iota = lax.iota(I32, 16)

        pltpu.sync_copy(wpack_ref, wbuf)
        for t in range(8):
            ones[pl.ds(t * 16, 16)] = jnp.full((16,), 1.0, F32)

        def process_graph(yz_ref, ei_ref, g, nreal, r, nchunks, ech, grp, out_row):
            base = sid * r
            nsl = pl.ds(base, r)
            lsl = pl.ds(0, r)
            ngrp = r // 16

            wvecs = [wbuf[pl.ds(112 * grp + 16 * t, 16)] for t in range(7)]

            def ws(n):
                return wvecs[n // 16][n % 16]

            w2l = [[ws(5 * i + j) for j in range(5)] for i in range(5)]
            w2r = [[ws(25 + 5 * i + j) for j in range(5)] for i in range(5)]
            b2 = [ws(50 + j) for j in range(5)]
            w3l = [ws(55 + 5 * i) + ws(56 + 5 * i) + ws(57 + 5 * i)
                   + ws(58 + 5 * i) + ws(59 + 5 * i) for i in range(5)]
            w3r = [ws(80 + 5 * i) + ws(81 + 5 * i) + ws(82 + 5 * i)
                   + ws(83 + 5 * i) + ws(84 + 5 * i) for i in range(5)]
            b3sum = ws(105) + ws(106) + ws(107) + ws(108) + ws(109)

            for c in range(5):
                pltpu.sync_copy(yz_ref.at[g, c, nsl], tb[c].at[nsl])
            for c in range(6):
                pltpu.sync_copy(z_ref.at[nsl], ac[c].at[nsl])
            eacc[...] = jnp.zeros((16,), F32)
            plsc.subcore_barrier()

            def edge_pass(ncols, with_deg):
                ebase = sid * nchunks

                def ck(k, carry):
                    pltpu.sync_copy(ei_ref.at[g, ebase + k], sidx)
                    pltpu.sync_copy(ei_ref.at[g, ech + ebase + k], didx)
                    descs = [pltpu.async_copy(tb[c].at[sidx], gr[c], sem)
                             for c in range(ncols)]
                    for d in descs:
                        d.wait()
                    descs = [pltpu.async_copy(gr[c], ac[c].at[didx], sem, add=True)
                             for c in range(ncols)]
                    if with_deg:
                        descs.append(pltpu.async_copy(ones, ac[5].at[didx], sem,
                                                      add=True))
                    for d in descs:
                        d.wait()
                    return carry

                lax.fori_loop(0, nchunks, ck, 0)

            edge_pass(5, True)
            plsc.subcore_barrier()

            for c in range(6):
                pltpu.sync_copy(ac[c].at[nsl], cb[c].at[lsl])
            for c in range(5):
                pltpu.sync_copy(yz_ref.at[g, 8 + c, nsl], zb[c].at[lsl])
                pltpu.sync_copy(z_ref.at[nsl], ac[c].at[nsl])

            def cgrp(i, carry):
                s16 = pl.ds(i * 16, 16)
                inv_v = 1.0 / jnp.maximum(cb[5][s16], 1.0)
                invb[s16] = inv_v
                h1 = [jnp.maximum(cb[c][s16] * inv_v + zb[c][s16], 0.0)
                      for c in range(5)]
                for j in range(5):
                    uj = h1[0] * w2l[0][j]
                    vj = h1[0] * w2r[0][j]
                    for c in range(1, 5):
                        uj = uj + h1[c] * w2l[c][j]
                        vj = vj + h1[c] * w2r[c][j]
                    ub[j][s16] = uj
                    vc[j][s16] = vj + b2[j]
                return carry

            lax.fori_loop(0, ngrp, cgrp, 0)
            for j in range(5):
                pltpu.sync_copy(ub[j].at[lsl], tb[j].at[nsl])
            plsc.subcore_barrier()

            edge_pass(5, False)
            plsc.subcore_barrier()

            for c in range(5):
                pltpu.sync_copy(ac[c].at[nsl], cb[c].at[lsl])
            pltpu.sync_copy(z_ref.at[nsl], ac[0].at[nsl])

            def dgrp(i, carry):
                s16 = pl.ds(i * 16, 16)
                gmask = (base + i * 16 + iota) < nreal
                inv_v = invb[s16]
                q = jnp.zeros((16,), F32)
                ec = jnp.zeros((16,), F32)
                for c in range(5):
                    h2c = jnp.maximum(cb[c][s16] * inv_v + vc[c][s16], 0.0)
                    q = q + h2c * w3l[c]
                    ec = ec + h2c * w3r[c]
                ub[0][s16] = jnp.where(gmask, q, 0.0)
                eacc[...] = eacc[...] + jnp.where(gmask, ec, 0.0)
                return carry

            lax.fori_loop(0, ngrp, dgrp, 0)
            pltpu.sync_copy(ub[0].at[lsl], tb[0].at[nsl])
            plsc.subcore_barrier()

            edge_pass(1, False)
            plsc.subcore_barrier()

            pltpu.sync_copy(ac[0].at[nsl], cb[0].at[lsl])

            def fgrp(i, carry):
                s16 = pl.ds(i * 16, 16)
                eacc[...] = eacc[...] + cb[0][s16] * invb[s16]
                return carry

            lax.fori_loop(0, ngrp, fgrp, 0)
            pltpu.sync_copy(eacc, part.at[pl.ds(sid * 16, 16)])
            plsc.subcore_barrier()

            @pl.when(sid == 0)
            def _():
                pltpu.sync_copy(part, partv)
                vtot = partv[pl.ds(0, 16)]
                for t in range(1, 16):
                    vtot = vtot + partv[pl.ds(t * 16, 16)]
                erow[...] = vtot + jnp.where(iota == 0, float(nreal) * b3sum, 0.0)
                pltpu.sync_copy(erow, out_ref.at[pl.ds(out_row * 16, 16)])

        @pl.when(cid == 0)
        def _big():
            for grp in range(2):
                def gb(gl, carry, grp=grp):
                    process_graph(yz_b_ref, ei_b_ref, grp * 2 + gl,
                                  10000, R_BIG, CH_BIG, EB // 128, grp,
                                  grp * 2 + gl)
                    return carry

                lax.fori_loop(0, 2, gb, 0)

        @pl.when(cid == 1)
        def _lig():
            def gl_(g, carry):
                process_graph(yz_l_ref, ei_l_ref, g,
                              2000, R_L, CH_L, EL // 128, 2, 4 + g)
                return carry

            lax.fori_loop(0, 20, gl_, 0)

    return body(yz_b, ei_b, yz_l, ei_l, wpack, zcol)


def _head_body(emat_ref, gcol_ref, grow_ref, bcol_ref, krow_ref,
               w1a, b1a, w2a, b2a, w3a, b3a,
               w1b, b1b, w2b, b2b, w3b, b3b,
               w1c, b1c, w2c, b2c, w3c, b3c, out_ref):
    emat = emat_ref[...]
    ecol = jnp.sum(emat, axis=1, keepdims=True)
    icol = lax.broadcasted_iota(I32, (24, 24), 0)
    irow = lax.broadcasted_iota(I32, (24, 24), 1)
    eye = (icol == irow).astype(F32)
    er = jnp.dot(jnp.ones((1, 24), F32), ecol * eye,
                 preferred_element_type=F32,
                 precision=lax.Precision.HIGHEST)
    lt = er < ecol
    eq = (er == ecol) & (irow < icol)
    sg = grow_ref[...] == gcol_ref[...]
    rank = jnp.sum(((lt | eq) & sg).astype(F32), axis=1, keepdims=True)
    pos = rank + bcol_ref[...]
    oh = (pos == krow_ref[...]).astype(F32)
    srt = jnp.dot(er, oh, preferred_element_type=F32,
                  precision=lax.Precision.HIGHEST)

    def mlp(v, w1, b1, w2, b2, w3, b3):
        hp = lax.Precision.HIGHEST
        h = jnp.maximum(jnp.dot(v, w1[...], preferred_element_type=F32,
                                precision=hp) + b1[...], 0.0)
        h = jnp.maximum(jnp.dot(h, w2[...], preferred_element_type=F32,
                                precision=hp) + b2[...], 0.0)
        return jnp.maximum(jnp.dot(h, w3[...], preferred_element_type=F32,
                                   precision=hp) + b3[...], 0.0)

    e_pl = mlp(srt[:, 0:2], w1a, b1a, w2a, b2a, w3a, b3a)
    e_p = mlp(srt[:, 2:4], w1b, b1b, w2b, b2b, w3b, b3b)
    e_l = mlp(srt[:, 4:24], w1c, b1c, w2c, b2c, w3c, b3c)
    out_ref[...] = e_pl - e_p - e_l


def _head(emat, gcol, grow, bcol, krow, mp):
    args = [emat, gcol, grow, bcol, krow]
    for k in ("pl", "p", "l"):
        p = mp[k]
        args += [p["W1"], p["b1"].reshape(1, -1), p["W2"], p["b2"].reshape(1, -1),
                 p["W3"], p["b3"].reshape(1, -1)]
    return pl.pallas_call(
        _head_body,
        out_shape=jax.ShapeDtypeStruct((1, 1), F32),
    )(*args)


def kernel(pl_x, pl_edge_index, pl_edge_attr, p_x, p_edge_index, p_edge_attr,
           l_x, l_edge_index, l_edge_attr, pl_params, p_params, l_params,
           mlp_params):
    x_big = jnp.pad(jnp.concatenate([pl_x, p_x], axis=0),
                    ((0, 0), (0, NB_BIG - pl_x.shape[1]), (0, 0)))
    ei_big = jnp.pad(jnp.concatenate([pl_edge_index, p_edge_index], axis=0),
                     ((0, 0), (0, 0), (0, EB - pl_edge_index.shape[2])),
                     constant_values=NB_BIG - 1).reshape(4, 2 * EB // 128, 128)
    x_l = jnp.pad(l_x, ((0, 0), (0, NB_L - l_x.shape[1]), (0, 0)))
    ei_l = jnp.pad(l_edge_index, ((0, 0), (0, 0), (0, EL - l_edge_index.shape[2])),
                   constant_values=NB_L - 1).reshape(20, 2 * EL // 128, 128)

    def wct(params):
        w1l, b1, w1r = params[0]
        wt = jnp.concatenate([w1l.T, jnp.zeros((3, 128), F32),
                              w1r.T, jnp.zeros((3, 128), F32)], axis=0)
        cc = jnp.concatenate([jnp.zeros(8, F32), b1, jnp.zeros(3, F32)])[:, None]
        return wt, cc

    wt_pl, cc_pl = wct(pl_params)
    wt_p, cc_p = wct(p_params)
    wt_l, cc_l = wct(l_params)
    wt_big = jnp.stack([wt_pl, wt_p])
    cc_big = jnp.stack([cc_pl, cc_p])

    def packw(params):
        (_, _, _), (w2l, b2, w2r), (w3l, b3, w3r) = params
        return jnp.concatenate([w2l.ravel(), w2r.ravel(), b2,
                                w3l.ravel(), w3r.ravel(), b3,
                                jnp.zeros(2, F32)])

    wpack = jnp.concatenate([packw(pl_params), packw(p_params), packw(l_params),
                             jnp.zeros(16, F32)])
    zcol = jnp.zeros((NB_BIG,), F32)

    yz_b = _proj(x_big, wt_big, cc_big, lambda g: g // 2, 2048)
    yz_l = _proj(x_l, wt_l[None], cc_l[None], lambda g: 0, 2048)

    energies = _sc_graphs(yz_b, ei_big, yz_l, ei_l, wpack, zcol).reshape(24, 16)

    gid = jnp.array([0.0, 0, 1, 1] + [2.0] * 20, F32)
    bas = jnp.array([0.0, 0, 2, 2] + [4.0] * 20, F32)
    krow = jnp.arange(24, dtype=F32)[None, :]
    out2 = _head(energies, gid[:, None], gid[None, :],
                 bas[:, None], krow, mlp_params)
    return out2.reshape((1,))
